# baseline (speedup 1.0000x reference)
import functools

import jax
import jax.numpy as jnp
from jax import lax
from jax.experimental import pallas as pl
from jax.experimental.pallas import tpu as pltpu
from jax.experimental.pallas import tpu_sc as plsc

V = 32
C = 128
KC = 8
ROUNDS = V // 2
L = 16


def _sc_body(counts_hbm, pc_hbm, out_hbm, rs_hbm, buf_a, buf_b, pc_v, rs_buf,
             acc_v, in_a, in_b, out_a, out_b):
    c = lax.axis_index("c")
    s = lax.axis_index("s")
    k0 = s * KC

    pltpu.sync_copy(pc_hbm, pc_v)
    pc_s = pc_v[...][0]

    def lane_total(vec):
        vals = [vec[i] for i in range(L)]
        while len(vals) > 1:
            vals = [a + b for a, b in zip(vals[::2], vals[1::2])]
        return vals[0]

    def in_slice(r):
        i = r * 2 + c
        return counts_hbm.at[i, :, pl.ds(k0, KC), :]

    def out_slice(r):
        i = r * 2 + c
        return out_hbm.at[i, :, pl.ds(k0, KC), :]

    lane_ids = lax.iota(jnp.int32, L)

    def compute(buf, r):
        rs_vec = jnp.zeros((L,), jnp.float32)
        for k in range(KC):
            def sum_j(j, acc):
                t = acc
                for lv in range(C // L):
                    t = t + buf[j, k, pl.ds(lv * L, L)]
                return t

            acc = lax.fori_loop(0, V, sum_j, jnp.zeros((L,), jnp.float32))
            rs_val = lane_total(acc) + pc_s * jnp.float32(V * C)
            rs_val_vec = jnp.full((L,), rs_val)
            denom = jnp.where(rs_val_vec > 0, rs_val_vec, jnp.float32(1.0))
            recip = jnp.full((L,), 1.0, jnp.float32) / denom
            pr = pc_v[...] * recip
            rs_vec = jnp.where(lane_ids == k, rs_val_vec, rs_vec)

            def scale_j(j, carry):
                for lv in range(C // L):
                    sl = pl.ds(lv * L, L)
                    buf[j, k, sl] = buf[j, k, sl] * recip + pr
                return carry

            lax.fori_loop(0, V, scale_j, 0)
        rs_buf[r] = rs_vec

    pltpu.async_copy(in_slice(0), buf_a, in_a)
    pltpu.async_copy(in_slice(1), buf_b, in_b)

    def round_pair(rr, carry):
        r0 = rr * 2
        r1 = r0 + 1
        pltpu.make_async_copy(in_slice(r0), buf_a, in_a).wait()
        compute(buf_a, r0)
        oa = pltpu.async_copy(buf_a, out_slice(r0), out_a)
        pltpu.make_async_copy(in_slice(r1), buf_b, in_b).wait()
        compute(buf_b, r1)
        ob = pltpu.async_copy(buf_b, out_slice(r1), out_b)

        @pl.when(rr < ROUNDS // 2 - 1)
        def _prefetch():
            oa.wait()
            pltpu.async_copy(in_slice(r0 + 2), buf_a, in_a)
            ob.wait()
            pltpu.async_copy(in_slice(r1 + 2), buf_b, in_b)

        return carry

    lax.fori_loop(0, ROUNDS // 2, round_pair, 0)
    pltpu.make_async_copy(buf_a, out_slice(ROUNDS - 2), out_a).wait()
    pltpu.make_async_copy(buf_b, out_slice(ROUNDS - 1), out_b).wait()
    pltpu.sync_copy(rs_buf, rs_hbm.at[:, c, s, :])


def kernel(transition_counts, pseudocount, hidden_states):
    del hidden_states
    counts = transition_counts.reshape(V, V, C, C)
    pcv = jnp.full((L,), pseudocount, jnp.float32)
    mesh = plsc.VectorSubcoreMesh(core_axis_name="c", subcore_axis_name="s")
    k = functools.partial(
        pl.kernel,
        mesh=mesh,
        out_type=[
            jax.ShapeDtypeStruct((V, V, C, C), jnp.float32),
            jax.ShapeDtypeStruct((ROUNDS, 2, V // 2, L), jnp.float32),
        ],
        scratch_types=[
            pltpu.VMEM((V, KC, C), jnp.float32),
            pltpu.VMEM((V, KC, C), jnp.float32),
            pltpu.VMEM((L,), jnp.float32),
            pltpu.VMEM((ROUNDS, L), jnp.float32),
            pltpu.VMEM((L,), jnp.float32),
            pltpu.SemaphoreType.DMA,
            pltpu.SemaphoreType.DMA,
            pltpu.SemaphoreType.DMA,
            pltpu.SemaphoreType.DMA,
        ],
    )(_sc_body)
    out, rs = k(counts, pcv)
    return out.reshape(-1), rs[..., :KC].reshape(-1)
